# baseline (speedup 1.0000x reference)
import jax
import jax.numpy as jnp
from jax import lax
from jax.experimental import pallas as pl
from jax.experimental.pallas import tpu as pltpu
from jax.experimental.pallas import tpu_sc as plsc

D0, D1, D2, K = 100000, 8, 64, 16384
P = D1 * D2
PH = P // 2
NC, NS, L = 2, 16, 16
NW = NC * NS
TC = 128
NTC = (D0 + TC - 1) // TC
TAILC = D0 - (NTC - 1) * TC
BASE_TCS = NTC // NW
EXTRA = NTC - BASE_TCS * NW
MAXCOLS = (BASE_TCS + 1) * TC
CAP = ((MAXCOLS + L) // L) * L + L
CHI = 2048
WB = 32
KB = 14
KM = (1 << KB) - 1


def _body(
    idx_hbm, upd_hbm, inpt_hbm, out_hbm,
    idxc, tbl, plist, bk, wub, slab0, slab1,
    sem, lsem0, lsem1, ssem0, ssem1,
):
    lsem = (lsem0, lsem1)
    ssem = (ssem0, ssem1)
    cid = lax.axis_index("c")
    sid = lax.axis_index("s")
    wid = sid * NC + cid
    lane = lax.iota(jnp.int32, L)

    tc0 = wid * BASE_TCS + jnp.minimum(wid, EXTRA)
    ntc = jnp.where(wid < EXTRA, BASE_TCS + 1, BASE_TCS)
    lo = tc0 * TC
    hi = jnp.minimum((tc0 + ntc) * TC, D0)

    @pl.loop(0, MAXCOLS // L)
    def _clear(g):
        tbl[pl.ds(g * L, L)] = jnp.full((L,), -1, jnp.int32)

    @pl.loop(0, K // CHI)
    def _chunk(ci):
        pltpu.sync_copy(idx_hbm.at[pl.ds(ci * CHI, CHI)], idxc)

        @pl.loop(0, CHI // L)
        def _scan(g):
            vidx = idxc[pl.ds(g * L, L)]
            mask = (vidx >= lo) & (vidx < hi)
            kvec = ci * CHI + g * L + lane
            addr = jnp.clip(vidx - lo, 0, MAXCOLS - 1)
            plsc.store_scatter(tbl, [addr], kvec, mask=mask)

            def _retry(active):
                got = plsc.load_gather(tbl, [addr], mask=mask)
                active = mask & (got < kvec)
                plsc.store_scatter(tbl, [addr], kvec, mask=active)
                return active

            lax.while_loop(jnp.any, _retry, mask)

    def _compress(g, cur):
        v = tbl[pl.ds(g * L, L)]
        m = v >= 0
        packed = ((lo + g * L + lane) << KB) | v
        plsc.store_compressed(plist.at[pl.ds(cur, L)], packed, mask=m)
        return cur + plsc.all_reduce_population_count(m)[0]

    cnt = lax.fori_loop(0, MAXCOLS // L, _compress, jnp.int32(0))
    nlv = (cnt + L - 1) // L

    slabs = (slab0, slab1)

    def _hslice(t, sh):
        off = pl.multiple_of(t * TC, TC)
        return (pl.ds(sh * PH, PH), pl.ds(off, TC))

    def _load(t, sh):
        s0, s1 = _hslice(t, sh)
        return pltpu.make_async_copy(
            inpt_hbm.at[s0, s1], slabs[sh], lsem[sh]
        )

    def _store(t, sh):
        s0, s1 = _hslice(t, sh)
        return pltpu.make_async_copy(
            slabs[sh], out_hbm.at[s0, s1], ssem[sh]
        )

    def _count(t):
        return lax.fori_loop(
            0,
            nlv,
            lambda j, a: a
            + plsc.all_reduce_population_count(
                ((j * L + lane) < cnt)
                & (
                    lax.shift_right_logical(plist[pl.ds(j * L, L)], KB)
                    < (t + 1) * TC
                )
            )[0],
            jnp.int32(0),
        )

    def _merge(t, i0, i1):
        nb = (i1 - i0 + WB - 1) // WB

        @pl.loop(0, nb)
        def _batch(b):
            b0 = i0 + b * WB
            for h in range(WB // L):
                v = plist[pl.ds(b0 + h * L, L)]
                bk[pl.ds(h * L, L)] = v & KM
            pltpu.async_copy(upd_hbm.at[bk], wub, sem).wait()
            nw = jnp.minimum(i1 - b0, WB)

            @pl.loop(0, nw)
            def _winner(j):
                v = plist[pl.ds(b0 + j, L)]
                rm = jnp.full((L,), (v[0] >> KB) - t * TC, jnp.int32)
                pos = jnp.full((L,), j, jnp.int32)
                for sh in range(2):
                    for g in range(PH // L):
                        cvec = g * L + lane
                        vals = plsc.load_gather(
                            wub, [pos, sh * PH + cvec]
                        )
                        plsc.store_scatter(slabs[sh], [cvec, rm], vals)

    def _pair(u, carry):
        i0 = carry
        t = tc0 + u
        i1 = _count(t)

        _load(t, 0).wait()
        _load(t, 1).wait()
        _merge(t, i0, i1)
        _store(t, 0).start()
        _store(t, 1).start()

        @pl.when(u + 1 < ntc)
        def _prefetch():
            _store(t, 0).wait()
            _load(t + 1, 0).start()
            _store(t, 1).wait()
            _load(t + 1, 1).start()

        return i1

    _load(tc0, 0).start()
    _load(tc0, 1).start()
    lax.fori_loop(0, ntc, _pair, jnp.int32(0))
    _store(tc0 + ntc - 1, 0).wait()
    _store(tc0 + ntc - 1, 1).wait()


_scatter_t = pl.kernel(
    _body,
    out_type=jax.ShapeDtypeStruct((P, D0), jnp.float32),
    mesh=plsc.VectorSubcoreMesh(core_axis_name="c", subcore_axis_name="s"),
    compiler_params=pltpu.CompilerParams(needs_layout_passes=False),
    scratch_types=[
        pltpu.VMEM((CHI,), jnp.int32),
        pltpu.VMEM((MAXCOLS,), jnp.int32),
        pltpu.VMEM((CAP,), jnp.int32),
        pltpu.VMEM((WB,), jnp.int32),
        pltpu.VMEM((WB, P), jnp.float32),
        pltpu.VMEM((PH, TC), jnp.float32),
        pltpu.VMEM((PH, TC), jnp.float32),
        pltpu.SemaphoreType.DMA,
        pltpu.SemaphoreType.DMA,
        pltpu.SemaphoreType.DMA,
        pltpu.SemaphoreType.DMA,
        pltpu.SemaphoreType.DMA,
    ],
)


def kernel(input, indices, update):
    inp_t = input.transpose(1, 2, 0).reshape(P, D0)
    upd_rows = update.reshape(K, P)
    out_t = _scatter_t(indices, upd_rows, inp_t)
    return out_t.reshape(D1, D2, D0).transpose(2, 0, 1)


# speedup vs baseline: 1.0959x; 1.0959x over previous
import jax
import jax.numpy as jnp
from jax import lax
from jax.experimental import pallas as pl
from jax.experimental.pallas import tpu as pltpu
from jax.experimental.pallas import tpu_sc as plsc

D0, D1, D2, K = 100000, 8, 64, 16384
P = D1 * D2
PH = P // 2
NC, NS, L = 2, 16, 16
NW = NC * NS
TC = 128
NTC = (D0 + TC - 1) // TC
TAILC = D0 - (NTC - 1) * TC
BASE_TCS = NTC // NW
EXTRA = NTC - BASE_TCS * NW
MAXCOLS = (BASE_TCS + 1) * TC
CAP = ((MAXCOLS + L) // L) * L + L
CHI = 2048
WB = 32
KB = 14
KM = (1 << KB) - 1


def _body(
    idx_hbm, upd_hbm, inpt_hbm, out_hbm,
    idxc, tbl, plist, bk, wub, slab0, slab1,
    sem, lsem0, lsem1, ssem0, ssem1,
):
    lsem = (lsem0, lsem1)
    ssem = (ssem0, ssem1)
    cid = lax.axis_index("c")
    sid = lax.axis_index("s")
    wid = sid * NC + cid
    lane = lax.iota(jnp.int32, L)

    tc0 = wid * BASE_TCS + jnp.minimum(wid, EXTRA)
    ntc = jnp.where(wid < EXTRA, BASE_TCS + 1, BASE_TCS)
    lo = tc0 * TC
    hi = jnp.minimum((tc0 + ntc) * TC, D0)

    @pl.loop(0, MAXCOLS // L)
    def _clear(g):
        tbl[pl.ds(g * L, L)] = jnp.full((L,), -1, jnp.int32)

    @pl.loop(0, K // CHI)
    def _chunk(ci):
        pltpu.sync_copy(idx_hbm.at[pl.ds(ci * CHI, CHI)], idxc)

        @pl.loop(0, CHI // L)
        def _scan(g):
            vidx = idxc[pl.ds(g * L, L)]
            mask = (vidx >= lo) & (vidx < hi)
            kvec = ci * CHI + g * L + lane
            addr = jnp.clip(vidx - lo, 0, MAXCOLS - 1)
            plsc.store_scatter(tbl, [addr], kvec, mask=mask)

            def _retry(active):
                got = plsc.load_gather(tbl, [addr], mask=mask)
                active = mask & (got < kvec)
                plsc.store_scatter(tbl, [addr], kvec, mask=active)
                return active

            lax.while_loop(jnp.any, _retry, mask)

    def _compress(g, cur):
        v = tbl[pl.ds(g * L, L)]
        m = v >= 0
        packed = ((lo + g * L + lane) << KB) | v
        plsc.store_compressed(plist.at[pl.ds(cur, L)], packed, mask=m)
        return cur + plsc.all_reduce_population_count(m)[0]

    cnt = lax.fori_loop(0, MAXCOLS // L, _compress, jnp.int32(0))
    nlv = (cnt + L - 1) // L

    slabs = (slab0, slab1)

    def _hslice(t, sh):
        off = pl.multiple_of(t * TC, TC)
        return (pl.ds(sh * PH, PH), pl.ds(off, TC))

    def _load(t, sh):
        s0, s1 = _hslice(t, sh)
        return pltpu.make_async_copy(
            inpt_hbm.at[s0, s1], slabs[sh], lsem[sh]
        )

    def _store(t, sh):
        s0, s1 = _hslice(t, sh)
        return pltpu.make_async_copy(
            slabs[sh], out_hbm.at[s0, s1], ssem[sh]
        )

    def _count(t):
        return lax.fori_loop(
            0,
            nlv,
            lambda j, a: a
            + plsc.all_reduce_population_count(
                ((j * L + lane) < cnt)
                & (
                    lax.shift_right_logical(plist[pl.ds(j * L, L)], KB)
                    < (t + 1) * TC
                )
            )[0],
            jnp.int32(0),
        )

    def _merge(t, i0, i1, sh):
        slab = slabs[sh]
        pbase = sh * PH
        nb = (i1 - i0 + WB - 1) // WB

        @pl.loop(0, nb)
        def _batch(b):
            b0 = i0 + b * WB
            for h in range(WB // L):
                v = plist[pl.ds(b0 + h * L, L)]
                bk[pl.ds(h * L, L)] = v & KM
            pltpu.async_copy(upd_hbm.at[bk], wub, sem).wait()
            nw = jnp.minimum(i1 - b0, WB)

            @pl.loop(0, nw)
            def _winner(j):
                v = plist[pl.ds(b0 + j, L)]
                rm = jnp.full((L,), (v[0] >> KB) - t * TC, jnp.int32)
                pos = jnp.full((L,), j, jnp.int32)
                for g in range(PH // L):
                    cvec = g * L + lane
                    vals = plsc.load_gather(wub, [pos, pbase + cvec])
                    plsc.store_scatter(slab, [cvec, rm], vals)

    def _pair(u, carry):
        i0 = carry
        t = tc0 + u
        i1 = _count(t)

        _load(t, 0).wait()
        _merge(t, i0, i1, 0)
        _store(t, 0).start()

        _load(t, 1).wait()
        _merge(t, i0, i1, 1)
        _store(t, 1).start()

        @pl.when(u + 1 < ntc)
        def _prefetch():
            _store(t, 0).wait()
            _load(t + 1, 0).start()
            _store(t, 1).wait()
            _load(t + 1, 1).start()

        return i1

    _load(tc0, 0).start()
    _load(tc0, 1).start()
    lax.fori_loop(0, ntc, _pair, jnp.int32(0))
    _store(tc0 + ntc - 1, 0).wait()
    _store(tc0 + ntc - 1, 1).wait()


_scatter_t = pl.kernel(
    _body,
    out_type=jax.ShapeDtypeStruct((P, D0), jnp.float32),
    mesh=plsc.VectorSubcoreMesh(core_axis_name="c", subcore_axis_name="s"),
    compiler_params=pltpu.CompilerParams(needs_layout_passes=False),
    scratch_types=[
        pltpu.VMEM((CHI,), jnp.int32),
        pltpu.VMEM((MAXCOLS,), jnp.int32),
        pltpu.VMEM((CAP,), jnp.int32),
        pltpu.VMEM((WB,), jnp.int32),
        pltpu.VMEM((WB, P), jnp.float32),
        pltpu.VMEM((PH, TC), jnp.float32),
        pltpu.VMEM((PH, TC), jnp.float32),
        pltpu.SemaphoreType.DMA,
        pltpu.SemaphoreType.DMA,
        pltpu.SemaphoreType.DMA,
        pltpu.SemaphoreType.DMA,
        pltpu.SemaphoreType.DMA,
    ],
)


def kernel(input, indices, update):
    inp_t = input.transpose(1, 2, 0).reshape(P, D0)
    upd_rows = update.reshape(K, P)
    out_t = _scatter_t(indices, upd_rows, inp_t)
    return out_t.reshape(D1, D2, D0).transpose(2, 0, 1)


# speedup vs baseline: 1.2147x; 1.1084x over previous
import jax
import jax.numpy as jnp
from jax import lax
from jax.experimental import pallas as pl
from jax.experimental.pallas import tpu as pltpu
from jax.experimental.pallas import tpu_sc as plsc

D0, D1, D2, K = 100000, 8, 64, 16384
P = D1 * D2
PH = P // 2
NC, NS, L = 2, 16, 16
NW = NC * NS
TC = 128
NTC = (D0 + TC - 1) // TC
TAILC = D0 - (NTC - 1) * TC
BASE_TCS = NTC // NW
EXTRA = NTC - BASE_TCS * NW
MAXCOLS = (BASE_TCS + 1) * TC
CAP = ((MAXCOLS + L) // L) * L + L
CHI = 2048
WB = 32
KB = 14
KM = (1 << KB) - 1


def _body(
    idx_hbm, upd_hbm, inpt_hbm, out_hbm,
    idxc, tbl, plist, bk, wub, slab0, slab1,
    sem, lsem0, lsem1, ssem0, ssem1,
):
    lsem = (lsem0, lsem1)
    ssem = (ssem0, ssem1)
    cid = lax.axis_index("c")
    sid = lax.axis_index("s")
    wid = sid * NC + cid
    lane = lax.iota(jnp.int32, L)

    tc0 = wid * BASE_TCS + jnp.minimum(wid, EXTRA)
    ntc = jnp.where(wid < EXTRA, BASE_TCS + 1, BASE_TCS)
    lo = tc0 * TC
    hi = jnp.minimum((tc0 + ntc) * TC, D0)

    @pl.loop(0, MAXCOLS // L)
    def _clear(g):
        tbl[pl.ds(g * L, L)] = jnp.full((L,), -1, jnp.int32)

    @pl.loop(0, K // CHI)
    def _chunk(ci):
        pltpu.sync_copy(idx_hbm.at[pl.ds(ci * CHI, CHI)], idxc)

        @pl.loop(0, CHI // L)
        def _scan(g):
            vidx = idxc[pl.ds(g * L, L)]
            mask = (vidx >= lo) & (vidx < hi)
            kvec = ci * CHI + g * L + lane
            addr = jnp.clip(vidx - lo, 0, MAXCOLS - 1)
            plsc.store_scatter(tbl, [addr], kvec, mask=mask)

            def _retry(active):
                got = plsc.load_gather(tbl, [addr], mask=mask)
                active = mask & (got < kvec)
                plsc.store_scatter(tbl, [addr], kvec, mask=active)
                return active

            lax.while_loop(jnp.any, _retry, mask)

    def _compress(g, cur):
        v = tbl[pl.ds(g * L, L)]
        m = v >= 0
        packed = ((lo + g * L + lane) << KB) | v
        plsc.store_compressed(plist.at[pl.ds(cur, L)], packed, mask=m)
        return cur + plsc.all_reduce_population_count(m)[0]

    cnt = lax.fori_loop(0, MAXCOLS // L, _compress, jnp.int32(0))
    nlv = (cnt + L - 1) // L

    slabs = (slab0, slab1)

    def _hslice(t, sh):
        off = pl.multiple_of(t * TC, TC)
        return (pl.ds(sh * PH, PH), pl.ds(off, TC))

    def _load(t, sh):
        s0, s1 = _hslice(t, sh)
        return pltpu.make_async_copy(
            inpt_hbm.at[s0, s1], slabs[sh], lsem[sh]
        )

    def _store(t, sh):
        s0, s1 = _hslice(t, sh)
        return pltpu.make_async_copy(
            slabs[sh], out_hbm.at[s0, s1], ssem[sh]
        )

    def _count(t):
        return lax.fori_loop(
            0,
            nlv,
            lambda j, a: a
            + plsc.all_reduce_population_count(
                ((j * L + lane) < cnt)
                & (
                    lax.shift_right_logical(plist[pl.ds(j * L, L)], KB)
                    < (t + 1) * TC
                )
            )[0],
            jnp.int32(0),
        )

    def _merge(t, i0, i1, sh, gather):
        slab = slabs[sh]
        pbase = sh * PH
        nb = (i1 - i0 + WB - 1) // WB

        @pl.loop(0, nb)
        def _batch(b):
            b0 = i0 + b * WB
            if gather:
                for h in range(WB // L):
                    v = plist[pl.ds(b0 + h * L, L)]
                    bk[pl.ds(h * L, L)] = v & KM
                pltpu.async_copy(upd_hbm.at[bk], wub, sem).wait()
            nw = jnp.minimum(i1 - b0, WB)

            @pl.loop(0, nw)
            def _winner(j):
                v = plist[pl.ds(b0 + j, L)]
                rm = jnp.full((L,), (v[0] >> KB) - t * TC, jnp.int32)
                pos = jnp.full((L,), j, jnp.int32)
                for g in range(PH // L):
                    cvec = g * L + lane
                    vals = plsc.load_gather(wub, [pos, pbase + cvec])
                    plsc.store_scatter(slab, [cvec, rm], vals)

    def _pair(u, carry):
        i0 = carry
        t = tc0 + u
        i1 = _count(t)

        _load(t, 0).wait()
        _merge(t, i0, i1, 0, True)
        _store(t, 0).start()

        _load(t, 1).wait()
        one_batch = (i1 - i0) <= WB

        @pl.when(one_batch)
        def _reuse():
            _merge(t, i0, i1, 1, False)

        @pl.when(jnp.logical_not(one_batch))
        def _regather():
            _merge(t, i0, i1, 1, True)

        _store(t, 1).start()

        @pl.when(u + 1 < ntc)
        def _prefetch():
            _store(t, 0).wait()
            _load(t + 1, 0).start()
            _store(t, 1).wait()
            _load(t + 1, 1).start()

        return i1

    _load(tc0, 0).start()
    _load(tc0, 1).start()
    lax.fori_loop(0, ntc, _pair, jnp.int32(0))
    _store(tc0 + ntc - 1, 0).wait()
    _store(tc0 + ntc - 1, 1).wait()


_scatter_t = pl.kernel(
    _body,
    out_type=jax.ShapeDtypeStruct((P, D0), jnp.float32),
    mesh=plsc.VectorSubcoreMesh(core_axis_name="c", subcore_axis_name="s"),
    compiler_params=pltpu.CompilerParams(needs_layout_passes=False),
    scratch_types=[
        pltpu.VMEM((CHI,), jnp.int32),
        pltpu.VMEM((MAXCOLS,), jnp.int32),
        pltpu.VMEM((CAP,), jnp.int32),
        pltpu.VMEM((WB,), jnp.int32),
        pltpu.VMEM((WB, P), jnp.float32),
        pltpu.VMEM((PH, TC), jnp.float32),
        pltpu.VMEM((PH, TC), jnp.float32),
        pltpu.SemaphoreType.DMA,
        pltpu.SemaphoreType.DMA,
        pltpu.SemaphoreType.DMA,
        pltpu.SemaphoreType.DMA,
        pltpu.SemaphoreType.DMA,
    ],
)


def kernel(input, indices, update):
    inp_t = input.transpose(1, 2, 0).reshape(P, D0)
    upd_rows = update.reshape(K, P)
    out_t = _scatter_t(indices, upd_rows, inp_t)
    return out_t.reshape(D1, D2, D0).transpose(2, 0, 1)


# speedup vs baseline: 1.2242x; 1.0078x over previous
import jax
import jax.numpy as jnp
from jax import lax
from jax.experimental import pallas as pl
from jax.experimental.pallas import tpu as pltpu
from jax.experimental.pallas import tpu_sc as plsc

D0, D1, D2, K = 100000, 8, 64, 16384
P = D1 * D2
PH = P // 2
NC, NS, L = 2, 16, 16
NW = NC * NS
TC = 128
NTC = (D0 + TC - 1) // TC
TAILC = D0 - (NTC - 1) * TC
BASE_TCS = NTC // NW
EXTRA = NTC - BASE_TCS * NW
MAXCOLS = (BASE_TCS + 1) * TC
CAP = ((MAXCOLS + L) // L) * L + L
CHI = K
WB = 32
KB = 14
KM = (1 << KB) - 1


def _body(
    idx_hbm, upd_hbm, inpt_hbm, out_hbm,
    idxc, tbl, plist, bk, wub, slab0, slab1,
    sem, lsem0, lsem1, ssem0, ssem1,
):
    lsem = (lsem0, lsem1)
    ssem = (ssem0, ssem1)
    cid = lax.axis_index("c")
    sid = lax.axis_index("s")
    wid = sid * NC + cid
    lane = lax.iota(jnp.int32, L)

    tc0 = wid * BASE_TCS + jnp.minimum(wid, EXTRA)
    ntc = jnp.where(wid < EXTRA, BASE_TCS + 1, BASE_TCS)
    lo = tc0 * TC
    hi = jnp.minimum((tc0 + ntc) * TC, D0)

    @pl.loop(0, MAXCOLS // L)
    def _clear(g):
        tbl[pl.ds(g * L, L)] = jnp.full((L,), -1, jnp.int32)

    pltpu.sync_copy(idx_hbm, idxc)

    @pl.loop(0, K // L)
    def _scan(g):
        vidx = idxc[pl.ds(g * L, L)]
        mask = (vidx >= lo) & (vidx < hi)
        kvec = g * L + lane
        addr = jnp.clip(vidx - lo, 0, MAXCOLS - 1)
        plsc.store_scatter(tbl, [addr], kvec, mask=mask)

        def _retry(active):
            got = plsc.load_gather(tbl, [addr], mask=mask)
            active = mask & (got < kvec)
            plsc.store_scatter(tbl, [addr], kvec, mask=active)
            return active

        lax.while_loop(jnp.any, _retry, mask)

    def _compress(g, cur):
        v = tbl[pl.ds(g * L, L)]
        m = v >= 0
        packed = ((lo + g * L + lane) << KB) | v
        plsc.store_compressed(plist.at[pl.ds(cur, L)], packed, mask=m)
        return cur + plsc.all_reduce_population_count(m)[0]

    cnt = lax.fori_loop(0, MAXCOLS // L, _compress, jnp.int32(0))
    nlv = (cnt + L - 1) // L

    slabs = (slab0, slab1)

    def _hslice(t, sh):
        off = pl.multiple_of(t * TC, TC)
        return (pl.ds(sh * PH, PH), pl.ds(off, TC))

    def _load(t, sh):
        s0, s1 = _hslice(t, sh)
        return pltpu.make_async_copy(
            inpt_hbm.at[s0, s1], slabs[sh], lsem[sh]
        )

    def _store(t, sh):
        s0, s1 = _hslice(t, sh)
        return pltpu.make_async_copy(
            slabs[sh], out_hbm.at[s0, s1], ssem[sh]
        )

    def _count(t):
        return lax.fori_loop(
            0,
            nlv,
            lambda j, a: a
            + plsc.all_reduce_population_count(
                ((j * L + lane) < cnt)
                & (
                    lax.shift_right_logical(plist[pl.ds(j * L, L)], KB)
                    < (t + 1) * TC
                )
            )[0],
            jnp.int32(0),
        )

    def _merge(t, i0, i1, sh, gather):
        slab = slabs[sh]
        pbase = sh * PH
        nb = (i1 - i0 + WB - 1) // WB

        @pl.loop(0, nb)
        def _batch(b):
            b0 = i0 + b * WB
            if gather:
                for h in range(WB // L):
                    v = plist[pl.ds(b0 + h * L, L)]
                    bk[pl.ds(h * L, L)] = v & KM
                pltpu.async_copy(upd_hbm.at[bk], wub, sem).wait()
            nw = jnp.minimum(i1 - b0, WB)

            @pl.loop(0, nw)
            def _winner(j):
                v = plist[pl.ds(b0 + j, L)]
                rm = jnp.full((L,), (v[0] >> KB) - t * TC, jnp.int32)
                pos = jnp.full((L,), j, jnp.int32)
                for g in range(PH // L):
                    cvec = g * L + lane
                    vals = plsc.load_gather(wub, [pos, pbase + cvec])
                    plsc.store_scatter(slab, [cvec, rm], vals)

    def _pair(u, carry):
        i0 = carry
        t = tc0 + u
        i1 = _count(t)

        _load(t, 0).wait()
        _merge(t, i0, i1, 0, True)
        _store(t, 0).start()

        _load(t, 1).wait()
        one_batch = (i1 - i0) <= WB

        @pl.when(one_batch)
        def _reuse():
            _merge(t, i0, i1, 1, False)

        @pl.when(jnp.logical_not(one_batch))
        def _regather():
            _merge(t, i0, i1, 1, True)

        _store(t, 1).start()

        @pl.when(u + 1 < ntc)
        def _prefetch():
            _store(t, 0).wait()
            _load(t + 1, 0).start()
            _store(t, 1).wait()
            _load(t + 1, 1).start()

        return i1

    _load(tc0, 0).start()
    _load(tc0, 1).start()
    lax.fori_loop(0, ntc, _pair, jnp.int32(0))
    _store(tc0 + ntc - 1, 0).wait()
    _store(tc0 + ntc - 1, 1).wait()


_scatter_t = pl.kernel(
    _body,
    out_type=jax.ShapeDtypeStruct((P, D0), jnp.float32),
    mesh=plsc.VectorSubcoreMesh(core_axis_name="c", subcore_axis_name="s"),
    compiler_params=pltpu.CompilerParams(needs_layout_passes=False),
    scratch_types=[
        pltpu.VMEM((CHI,), jnp.int32),
        pltpu.VMEM((MAXCOLS,), jnp.int32),
        pltpu.VMEM((CAP,), jnp.int32),
        pltpu.VMEM((WB,), jnp.int32),
        pltpu.VMEM((WB, P), jnp.float32),
        pltpu.VMEM((PH, TC), jnp.float32),
        pltpu.VMEM((PH, TC), jnp.float32),
        pltpu.SemaphoreType.DMA,
        pltpu.SemaphoreType.DMA,
        pltpu.SemaphoreType.DMA,
        pltpu.SemaphoreType.DMA,
        pltpu.SemaphoreType.DMA,
    ],
)


def kernel(input, indices, update):
    inp_t = input.transpose(1, 2, 0).reshape(P, D0)
    upd_rows = update.reshape(K, P)
    out_t = _scatter_t(indices, upd_rows, inp_t)
    return out_t.reshape(D1, D2, D0).transpose(2, 0, 1)
